# baseline (speedup 1.0000x reference)
import functools

import jax
import jax.numpy as jnp
from jax import lax
from jax.experimental import pallas as pl
from jax.experimental.pallas import tpu as pltpu
from jax.experimental.pallas import tpu_sc as plsc

N = 50000
E = 800000
G = 256
D = 256
F32 = jnp.float32
I32 = jnp.int32

RB = 2000
NBLK = N // RB

NC = 2
NS = 16
L = 16
NW = NC * NS

EPG = 25088
ETOT = NW * EPG
SBB = 512
NSB_BIN = EPG // SBB
SB = 128
NPAD = 53248
NZ = 13
CHUNK = 4096
ACC_ROWS = 136
PAD_COL = NPAD - 1


def _embed_body(x_ref, w_ref, b_ref, o_ref):
    o_ref[...] = (
        jnp.dot(x_ref[...], w_ref[...], preferred_element_type=F32) + b_ref[...]
    )


def _tc_embed(x, w, b):
    return pl.pallas_call(
        _embed_body,
        grid=(NBLK,),
        in_specs=[
            pl.BlockSpec((RB, 128), lambda i: (i, 0)),
            pl.BlockSpec((128, D), lambda i: (0, 0)),
            pl.BlockSpec((1, D), lambda i: (0, 0)),
        ],
        out_specs=pl.BlockSpec((RB, D), lambda i: (i, 0)),
        out_shape=jax.ShapeDtypeStruct((N, D), F32),
    )(x, w, b)


def _uc_body(m0_ref, badd_ref, w3_ref, uc_ref):
    m = m0_ref[...]
    badd = badd_ref[...]
    w3 = w3_ref[...]
    for k in range(4):
        m = jnp.dot(m, w3, preferred_element_type=F32) + badd
        uc_ref[pl.ds(2 * k, 2), :] = m[0:2, :]


def _tc_uc(m0, badd, w3):
    return pl.pallas_call(
        _uc_body,
        grid=(1,),
        in_specs=[
            pl.BlockSpec((8, D), lambda i: (0, 0)),
            pl.BlockSpec((8, D), lambda i: (0, 0)),
            pl.BlockSpec((D, D), lambda i: (0, 0)),
        ],
        out_specs=pl.BlockSpec((8, D), lambda i: (0, 0)),
        out_shape=jax.ShapeDtypeStruct((8, D), F32),
    )(m0, badd, w3)


def _tables_first_body(x_ref, w1_ref, w2_ref, c1_ref, p_ref, a_ref, b_ref):
    x = x_ref[...]
    p1 = jnp.dot(x, w1_ref[...], preferred_element_type=F32)
    q1 = jnp.dot(x, w2_ref[...], preferred_element_type=F32)
    p_ref[...] = p1
    a_ref[...] = x + p1
    b_ref[...] = q1 + c1_ref[...]


def _tc_tables_first(x, w1, w2, c1):
    full = lambda s: pl.BlockSpec(s, lambda i: (0, 0))
    blk = pl.BlockSpec((RB, D), lambda i: (i, 0))
    sds = jax.ShapeDtypeStruct((N, D), F32)
    sds_b = jax.ShapeDtypeStruct((NPAD, D), F32)
    return pl.pallas_call(
        _tables_first_body,
        grid=(NBLK,),
        in_specs=[blk, full((D, D)), full((D, D)), full((1, D))],
        out_specs=(blk, blk, blk),
        out_shape=(sds, sds, sds_b),
    )(x, w1, w2, c1)


def _tables_body(h_ref, st_ref, p_ref, bp_ref, w1_ref, w2_ref, w3_ref,
                 ci_ref, cn_ref, po_ref, a_ref, b_ref):
    st = st_ref[...]
    x = jnp.maximum(h_ref[...] * st[0:1, :] + st[1:2, :], 0.0)
    q = bp_ref[...] - ci_ref[...]
    pn = (jnp.dot(x, w1_ref[...], preferred_element_type=F32)
          + jnp.dot(p_ref[...], w3_ref[...], preferred_element_type=F32))
    qn = (jnp.dot(x, w2_ref[...], preferred_element_type=F32)
          + jnp.dot(q, w3_ref[...], preferred_element_type=F32))
    po_ref[...] = pn
    a_ref[...] = x + pn
    b_ref[...] = qn + cn_ref[...]


def _tc_tables(h, st, p, bprev, w1, w2, w3, ci, cnext):
    full = lambda s: pl.BlockSpec(s, lambda i: (0, 0))
    blk = pl.BlockSpec((RB, D), lambda i: (i, 0))
    sds = jax.ShapeDtypeStruct((N, D), F32)
    sds_b = jax.ShapeDtypeStruct((NPAD, D), F32)
    return pl.pallas_call(
        _tables_body,
        grid=(NBLK,),
        in_specs=[blk, full((8, D)), blk, blk, full((D, D)), full((D, D)),
                  full((D, D)), full((1, D)), full((1, D))],
        out_specs=(blk, blk, blk),
        out_shape=(sds, sds, sds_b),
    )(h, st, p, bprev, w1, w2, w3, ci, cnext)


def _post_body(agg_ref, xs_ref, st_ref, w_ref, b_ref, h_ref, s1_ref, s2_ref,
               *, act):
    if act:
        st = st_ref[...]
        x = jnp.maximum(xs_ref[...] * st[0:1, :] + st[1:2, :], 0.0)
    else:
        x = xs_ref[...]
    m = agg_ref[...] + x
    h = jnp.dot(m, w_ref[...], preferred_element_type=F32) + b_ref[...]
    h_ref[...] = h

    @pl.when(pl.program_id(0) == 0)
    def _():
        s1_ref[...] = jnp.zeros_like(s1_ref)
        s2_ref[...] = jnp.zeros_like(s2_ref)

    s1b = jnp.sum(h, axis=0, keepdims=True)
    mb = s1b * F32(1.0 / RB)
    hc = h - mb
    s1_ref[0:1, :] += s1b
    s2_ref[0:1, :] += jnp.sum(hc * hc, axis=0, keepdims=True)
    s2_ref[1:2, :] += mb * mb


def _tc_post(aggp, xsrc, st, w, b, act):
    full = lambda s: pl.BlockSpec(s, lambda i: (0, 0))
    blk = pl.BlockSpec((RB, D), lambda i: (i, 0))
    return pl.pallas_call(
        functools.partial(_post_body, act=act),
        grid=(NBLK,),
        in_specs=[blk, blk, full((8, D)), full((D, D)), full((1, D))],
        out_specs=(blk, full((8, D)), full((8, D))),
        out_shape=(jax.ShapeDtypeStruct((N, D), F32),
                   jax.ShapeDtypeStruct((8, D), F32),
                   jax.ShapeDtypeStruct((8, D), F32)),
    )(aggp, xsrc, st, w, b)


def _stats_body(s1_ref, s2_ref, g_ref, bb_ref, st_ref):
    inv_n = F32(1.0 / N)
    mean = s1_ref[0:1, :] * inv_n
    var = (s2_ref[0:1, :] + F32(RB) * s2_ref[1:2, :]) * inv_n - mean * mean
    s = g_ref[...] * lax.rsqrt(var + 1e-5)
    t = bb_ref[...] - mean * s
    st_ref[...] = jnp.zeros_like(st_ref)
    st_ref[0:1, :] = s
    st_ref[1:2, :] = t


def _tc_stats(s1, s2, gamma, beta):
    full = lambda s: pl.BlockSpec(s, lambda i: (0, 0))
    return pl.pallas_call(
        _stats_body,
        grid=(1,),
        in_specs=[full((8, D)), full((8, D)), full((1, D)), full((1, D))],
        out_specs=full((8, D)),
        out_shape=jax.ShapeDtypeStruct((8, D), F32),
    )(s1, s2, gamma, beta)


def _pool_body(h_ref, st_ref, bid_ref, o_ref):
    st = st_ref[...]
    x = jnp.maximum(h_ref[...] * st[0:1, :] + st[1:2, :], 0.0)
    bid = bid_ref[0, 0, :]
    onehot = (lax.broadcasted_iota(I32, (RB, G), 1) == bid[:, None]).astype(F32)
    part = lax.dot_general(onehot, x, (((0,), (0,)), ((), ())),
                           preferred_element_type=F32)

    @pl.when(pl.program_id(0) == 0)
    def _():
        o_ref[...] = jnp.zeros_like(o_ref)

    o_ref[...] += part


def _tc_pool(h, st, bid3):
    full = lambda s: pl.BlockSpec(s, lambda i: (0, 0))
    return pl.pallas_call(
        _pool_body,
        grid=(NBLK,),
        in_specs=[pl.BlockSpec((RB, D), lambda i: (i, 0)), full((8, D)),
                  pl.BlockSpec((1, 1, RB), lambda i: (i, 0, 0))],
        out_specs=full((G, D)),
        out_shape=jax.ShapeDtypeStruct((G, D), F32),
    )(h, st, bid3)


def _out_body(p_ref, w_ref, b_ref, o_ref):
    o_ref[...] = (
        jnp.dot(p_ref[...], w_ref[...], preferred_element_type=F32) + b_ref[...]
    )


def _tc_out(pooled, w, b):
    full = lambda s: pl.BlockSpec(s, lambda i: (0, 0))
    return pl.pallas_call(
        _out_body,
        grid=(1,),
        in_specs=[full((G, D)), full((D, 128)), full((1, 128))],
        out_specs=full((G, 128)),
        out_shape=jax.ShapeDtypeStruct((G, 128), F32),
    )(pooled, w, b)


def _sc_mesh():
    return plsc.VectorSubcoreMesh(
        core_axis_name="c", subcore_axis_name="s",
        num_cores=NC, num_subcores=NS)


def _lane_i32(vec, lane, iota):
    return jnp.sum(jnp.where(iota == lane, vec, 0))


def _bin_body(row_h, col_h, w_h, brow_h, bcol_h, bw_h, offs_h,
              arow_h, acol_h, aw_h,
              brow_v, bcol_v, bw_v, inrow_v, incol_v, inw_v, ov_v):
    cid = lax.axis_index("c")
    sid = lax.axis_index("s")
    wid = sid * NC + cid
    base = wid * EPG
    iota = lax.iota(I32, L)

    def owner_body(c, carry):
        cursor, ov0, ov1 = carry
        ov0 = jnp.where(iota == c, cursor, ov0)
        ov1 = jnp.where(iota + L == c, cursor, ov1)

        def sb_body(sb, cur):
            off = pl.multiple_of(base + sb * SBB, 16)
            pltpu.sync_copy(row_h.at[pl.ds(off, SBB)], inrow_v)
            pltpu.sync_copy(col_h.at[pl.ds(off, SBB)], incol_v)
            pltpu.sync_copy(w_h.at[pl.ds(off, SBB)], inw_v)
            for j in range(SBB // L):
                cv = incol_v[pl.ds(j * L, L)]
                mask = ((cv >> 6) & 31) == c
                ones = jnp.where(mask, 1, 0)
                dest = cur + plsc.cumsum(ones) - 1
                plsc.store_scatter(brow_v, [dest],
                                   inrow_v[pl.ds(j * L, L)], mask=mask)
                plsc.store_scatter(bcol_v, [dest], cv, mask=mask)
                plsc.store_scatter(bw_v, [dest],
                                   inw_v[pl.ds(j * L, L)], mask=mask)
                cur = cur + jnp.sum(ones)
            return cur

        cursor = lax.fori_loop(0, NSB_BIN, sb_body, cursor)
        return cursor, ov0, ov1

    init = (I32(0), jnp.full((L,), EPG, I32), jnp.full((L,), EPG, I32))
    _, ov0, ov1 = lax.fori_loop(0, NW, owner_body, init)

    pltpu.sync_copy(brow_v.at[pl.ds(0, EPG)], arow_h.at[pl.ds(base, EPG)])
    pltpu.sync_copy(bcol_v.at[pl.ds(0, EPG)], acol_h.at[pl.ds(base, EPG)])
    pltpu.sync_copy(bw_v.at[pl.ds(0, EPG)], aw_h.at[pl.ds(base, EPG)])

    def seg_body(o, cursor):
        lane_a = o & 15
        vsel_a = o >> 4
        s_a = _lane_i32(ov0, lane_a, iota)
        s_b = _lane_i32(ov1, lane_a, iota)
        seg_lo = jnp.where(vsel_a == 0, s_a, s_b)
        o2 = o + 1
        lane_b = o2 & 15
        vsel_b = o2 >> 4
        e_a = _lane_i32(ov0, lane_b, iota)
        e_b = _lane_i32(ov1, lane_b, iota)
        seg_hi = jnp.where(vsel_b == 0, e_a,
                           jnp.where(vsel_b == 1, e_b, EPG))
        astart = seg_lo & ~15
        span = seg_hi - astart
        nsb = (span + SBB - 1) // SBB

        def z_body(z, carry):
            cur, ovz = carry
            ovz = jnp.where(iota == z, cur, ovz)

            def sb_body(sb, cur2):
                abase = pl.multiple_of(astart + sb * SBB, 16)
                off = pl.multiple_of(base + abase, 16)
                pltpu.sync_copy(arow_h.at[pl.ds(off, SBB)], inrow_v)
                pltpu.sync_copy(acol_h.at[pl.ds(off, SBB)], incol_v)
                pltpu.sync_copy(aw_h.at[pl.ds(off, SBB)], inw_v)
                for j in range(SBB // L):
                    pos = abase + j * L + iota
                    valid = (pos >= seg_lo) & (pos < seg_hi)
                    cv = incol_v[pl.ds(j * L, L)]
                    mask = valid & ((cv >> 12) == z)
                    ones = jnp.where(mask, 1, 0)
                    dest = cur2 + plsc.cumsum(ones) - 1
                    plsc.store_scatter(brow_v, [dest],
                                       inrow_v[pl.ds(j * L, L)], mask=mask)
                    plsc.store_scatter(bcol_v, [dest], cv, mask=mask)
                    plsc.store_scatter(bw_v, [dest],
                                       inw_v[pl.ds(j * L, L)], mask=mask)
                    cur2 = cur2 + jnp.sum(ones)
                return cur2

            cur = lax.fori_loop(0, nsb, sb_body, cur)
            return cur, ovz

        cursor, ovz = lax.fori_loop(
            0, NZ, z_body, (cursor, jnp.full((L,), EPG, I32)))
        ovz = jnp.where(iota == NZ, cursor, ovz)
        ov_v[pl.ds(0, L)] = ovz
        dst = pl.multiple_of((wid * NW + o) * L, 16)
        pltpu.sync_copy(ov_v.at[pl.ds(0, L)], offs_h.at[pl.ds(dst, L)])
        return cursor

    lax.fori_loop(0, NW, seg_body, I32(0))

    pltpu.sync_copy(brow_v.at[pl.ds(0, EPG)], brow_h.at[pl.ds(base, EPG)])
    pltpu.sync_copy(bcol_v.at[pl.ds(0, EPG)], bcol_h.at[pl.ds(base, EPG)])
    pltpu.sync_copy(bw_v.at[pl.ds(0, EPG)], bw_h.at[pl.ds(base, EPG)])


def _sc_bin(row_p, col_p, w_p):
    f = pl.kernel(
        _bin_body,
        out_type=(
            jax.ShapeDtypeStruct((ETOT + SB,), I32),
            jax.ShapeDtypeStruct((ETOT + SB,), I32),
            jax.ShapeDtypeStruct((ETOT + SB,), F32),
            jax.ShapeDtypeStruct((NW * NW * L,), I32),
            jax.ShapeDtypeStruct((ETOT + SBB,), I32),
            jax.ShapeDtypeStruct((ETOT + SBB,), I32),
            jax.ShapeDtypeStruct((ETOT + SBB,), F32),
        ),
        mesh=_sc_mesh(),
        compiler_params=pltpu.CompilerParams(needs_layout_passes=False),
        scratch_types=[
            pltpu.VMEM((EPG + L,), I32),
            pltpu.VMEM((EPG + L,), I32),
            pltpu.VMEM((EPG + L,), F32),
            pltpu.VMEM((SBB,), I32),
            pltpu.VMEM((SBB,), I32),
            pltpu.VMEM((SBB,), F32),
            pltpu.VMEM((L,), I32),
        ],
    )
    return f(row_p, col_p, w_p)


def _scatter_body(a_h, b_h, u_h, brow_h, bcol_h, bw_h, offs_h, agg_h,
                  acc, mbuf, bwin, vrow, vcol, vw, rowidx, relbuf,
                  u_v, offs_v, sem_a, sem_b):
    cid = lax.axis_index("c")
    sid = lax.axis_index("s")
    wid = sid * NC + cid
    iota = lax.iota(I32, L)

    pltpu.sync_copy(u_h, u_v)
    u_regs = [u_v[pl.ds(j * L, L)] for j in range(D // L)]
    lane_j = [j * L + iota for j in range(D // L)]

    def z_body(z, _):
        def zr(r, _):
            for j in range(D // L):
                acc[r, pl.ds(j * L, L)] = jnp.zeros((L,), F32)
            return 0

        lax.fori_loop(0, ACC_ROWS, zr, 0)

        for s in range(2):
            srcb = pl.multiple_of(z * CHUNK + s * 2048 + wid * 64, 64)
            pltpu.sync_copy(b_h.at[pl.ds(srcb, 64)],
                            bwin.at[pl.ds(s * 64, 64)])

        def tile_body(t, _):
            src = pl.multiple_of((t * NW + wid) * L, 16)
            pltpu.sync_copy(offs_h.at[pl.ds(src, L)], offs_v)
            ov = offs_v[...]
            start = _lane_i32(ov, z, iota)
            end = _lane_i32(ov, z + 1, iota)
            count = end - start
            astart = start & ~15
            span = start + count - astart
            nsb = (span + SB - 1) // SB
            base_e = t * EPG

            def sb_body(sb, _):
                abase = pl.multiple_of(astart + sb * SB, 16)
                pltpu.sync_copy(brow_h.at[pl.ds(base_e + abase, SB)], vrow)
                pltpu.sync_copy(bcol_h.at[pl.ds(base_e + abase, SB)], vcol)
                pltpu.sync_copy(bw_h.at[pl.ds(base_e + abase, SB)], vw)
                for j in range(SB // L):
                    pos = abase + j * L + iota
                    valid = (pos >= start) & (pos < start + count)
                    cv = vcol[pl.ds(j * L, L)]
                    rowidx[pl.ds(j * L, L)] = jnp.where(
                        valid, vrow[pl.ds(j * L, L)], 0)
                    relbuf[pl.ds(j * L, L)] = ((cv >> 11) & 1) * 64 + (cv & 63)
                pltpu.async_copy(a_h.at[rowidx], mbuf, sem_a).wait()

                e_lo = jnp.maximum(start - abase, 0)
                e_hi = jnp.minimum(start + count - abase, SB)

                def e_body(e, _):
                    e16 = jnp.full((L,), e, I32)
                    we = plsc.load_gather(vw, [e16])
                    rel16 = plsc.load_gather(relbuf, [e16])
                    for j in range(D // L):
                        a = mbuf[e, pl.ds(j * L, L)]
                        b = plsc.load_gather(bwin, [rel16, lane_j[j]])
                        msg = jnp.maximum(a + b + we * u_regs[j], 0.0)
                        plsc.addupdate_scatter(acc, [rel16, lane_j[j]], msg)
                    return 0

                lax.fori_loop(e_lo, e_hi, e_body, 0)
                return 0

            lax.fori_loop(0, nsb, sb_body, 0)
            return 0

        lax.fori_loop(0, NW, tile_body, 0)

        for s in range(2):
            dsts = z * CHUNK + s * 2048 + wid * 64
            dst = pl.multiple_of(dsts, 64)
            pltpu.sync_copy(acc.at[pl.ds(s * 64, 64)],
                            agg_h.at[pl.ds(dst, 64)])
        return 0

    lax.fori_loop(0, NZ, z_body, 0)


def _sc_scatter(a, b, u, brow, bcol, bw, offs):
    f = pl.kernel(
        _scatter_body,
        out_type=jax.ShapeDtypeStruct((NPAD, D), F32),
        mesh=_sc_mesh(),
        compiler_params=pltpu.CompilerParams(needs_layout_passes=False),
        scratch_types=[
            pltpu.VMEM((ACC_ROWS, D), F32),
            pltpu.VMEM((SB, D), F32),
            pltpu.VMEM((128, D), F32),
            pltpu.VMEM((SB,), I32),
            pltpu.VMEM((SB,), I32),
            pltpu.VMEM((SB,), F32),
            pltpu.VMEM((SB,), I32),
            pltpu.VMEM((SB,), I32),
            pltpu.VMEM((D,), F32),
            pltpu.VMEM((L,), I32),
            pltpu.SemaphoreType.DMA,
            pltpu.SemaphoreType.DMA,
        ],
    )
    return f(a, b, u, brow, bcol, bw, offs)


def kernel(node_features, edge_index, atom_weights, batch, atom_W, atom_b,
           edge_W, edge_b, mlp_W, mlp_b, nn_W, nn_b, bn_gamma, bn_beta,
           out_W, out_b):
    nf_p = jnp.pad(node_features, ((0, 0), (0, 128 - node_features.shape[1])))
    atom_W_p = jnp.pad(atom_W, ((0, 128 - atom_W.shape[0]), (0, 0)))
    row = edge_index[0].astype(I32)
    col = edge_index[1].astype(I32)
    w_e = atom_weights[:, 0]
    row_p = jnp.pad(row, (0, ETOT - E))
    col_p = jnp.pad(col, (0, ETOT - E), constant_values=PAD_COL)
    w_p = jnp.pad(w_e, (0, ETOT - E))
    w1 = mlp_W[0:D]
    w2 = mlp_W[D:2 * D]
    w3 = mlp_W[2 * D:3 * D]
    m0 = jnp.concatenate([edge_W, edge_b[None, :], jnp.zeros((6, D), F32)], 0)
    badd = jnp.concatenate(
        [jnp.zeros((1, D), F32), mlp_b[None, :], jnp.zeros((6, D), F32)], 0)
    bid3 = batch.astype(I32).reshape(NBLK, 1, RB)
    out_W_p = jnp.pad(out_W, ((0, 0), (0, 128 - out_W.shape[1])))
    out_b_p = jnp.pad(out_b, (0, 128 - out_b.shape[0]))[None, :]

    x0 = _tc_embed(nf_p, atom_W_p, atom_b[None, :])
    uc = _tc_uc(m0, badd, w3)
    brow, bcol, bw, offs, _, _, _ = _sc_bin(row_p, col_p, w_p)

    p, a, b = _tc_tables_first(x0, w1, w2, uc[1:2])
    agg = _sc_scatter(a, b, uc[0], brow, bcol, bw, offs)
    h, s1, s2 = _tc_post(agg, x0, uc, nn_W, nn_b[None, :], act=False)
    st = _tc_stats(s1, s2, bn_gamma[None, :], bn_beta[None, :])

    for i in (1, 2, 3):
        ci = uc[2 * i - 1:2 * i]
        cn = uc[2 * i + 1:2 * i + 2]
        p, a, b = _tc_tables(h, st, p, b, w1, w2, w3, ci, cn)
        agg = _sc_scatter(a, b, uc[2 * i], brow, bcol, bw, offs)
        h, s1, s2 = _tc_post(agg, h, st, nn_W, nn_b[None, :], act=True)
        st = _tc_stats(s1, s2, bn_gamma[None, :], bn_beta[None, :])

    pooled = _tc_pool(h, st, bid3)
    out = _tc_out(pooled, out_W_p, out_b_p)
    return out[:, :6]


# speedup vs baseline: 2.4220x; 2.4220x over previous
import functools

import jax
import jax.numpy as jnp
from jax import lax
from jax.experimental import pallas as pl
from jax.experimental.pallas import tpu as pltpu
from jax.experimental.pallas import tpu_sc as plsc

N = 50000
E = 800000
G = 256
D = 256
F32 = jnp.float32
I32 = jnp.int32

RB = 2000
NBLK = N // RB

NC = 2
NS = 16
L = 16
NW = NC * NS

EPG = 25088
ETOT = NW * EPG
SBB = 512
NSB_BIN = EPG // SBB
SB = 32
NPAD = 53248
NZ = 13
CHUNK = 4096
ACC_ROWS = 136
PAD_COL = NPAD - 1


def _embed_body(x_ref, w_ref, b_ref, o_ref):
    o_ref[...] = (
        jnp.dot(x_ref[...], w_ref[...], preferred_element_type=F32) + b_ref[...]
    )


def _tc_embed(x, w, b):
    return pl.pallas_call(
        _embed_body,
        grid=(NBLK,),
        in_specs=[
            pl.BlockSpec((RB, 128), lambda i: (i, 0)),
            pl.BlockSpec((128, D), lambda i: (0, 0)),
            pl.BlockSpec((1, D), lambda i: (0, 0)),
        ],
        out_specs=pl.BlockSpec((RB, D), lambda i: (i, 0)),
        out_shape=jax.ShapeDtypeStruct((N, D), F32),
    )(x, w, b)


def _uc_body(m0_ref, badd_ref, w3_ref, uc_ref):
    m = m0_ref[...]
    badd = badd_ref[...]
    w3 = w3_ref[...]
    for k in range(4):
        m = jnp.dot(m, w3, preferred_element_type=F32) + badd
        uc_ref[pl.ds(2 * k, 2), :] = m[0:2, :]


def _tc_uc(m0, badd, w3):
    return pl.pallas_call(
        _uc_body,
        grid=(1,),
        in_specs=[
            pl.BlockSpec((8, D), lambda i: (0, 0)),
            pl.BlockSpec((8, D), lambda i: (0, 0)),
            pl.BlockSpec((D, D), lambda i: (0, 0)),
        ],
        out_specs=pl.BlockSpec((8, D), lambda i: (0, 0)),
        out_shape=jax.ShapeDtypeStruct((8, D), F32),
    )(m0, badd, w3)


def _tables_first_body(x_ref, w1_ref, w2_ref, c1_ref, p_ref, a_ref, b_ref):
    x = x_ref[...]
    p1 = jnp.dot(x, w1_ref[...], preferred_element_type=F32)
    q1 = jnp.dot(x, w2_ref[...], preferred_element_type=F32)
    p_ref[...] = p1
    a_ref[...] = x + p1
    b_ref[...] = q1 + c1_ref[...]


def _tc_tables_first(x, w1, w2, c1):
    full = lambda s: pl.BlockSpec(s, lambda i: (0, 0))
    blk = pl.BlockSpec((RB, D), lambda i: (i, 0))
    sds = jax.ShapeDtypeStruct((N, D), F32)
    sds_b = jax.ShapeDtypeStruct((NPAD, D), F32)
    return pl.pallas_call(
        _tables_first_body,
        grid=(NBLK,),
        in_specs=[blk, full((D, D)), full((D, D)), full((1, D))],
        out_specs=(blk, blk, blk),
        out_shape=(sds, sds, sds_b),
    )(x, w1, w2, c1)


def _tables_body(h_ref, st_ref, p_ref, bp_ref, w1_ref, w2_ref, w3_ref,
                 ci_ref, cn_ref, po_ref, a_ref, b_ref):
    st = st_ref[...]
    x = jnp.maximum(h_ref[...] * st[0:1, :] + st[1:2, :], 0.0)
    q = bp_ref[...] - ci_ref[...]
    pn = (jnp.dot(x, w1_ref[...], preferred_element_type=F32)
          + jnp.dot(p_ref[...], w3_ref[...], preferred_element_type=F32))
    qn = (jnp.dot(x, w2_ref[...], preferred_element_type=F32)
          + jnp.dot(q, w3_ref[...], preferred_element_type=F32))
    po_ref[...] = pn
    a_ref[...] = x + pn
    b_ref[...] = qn + cn_ref[...]


def _tc_tables(h, st, p, bprev, w1, w2, w3, ci, cnext):
    full = lambda s: pl.BlockSpec(s, lambda i: (0, 0))
    blk = pl.BlockSpec((RB, D), lambda i: (i, 0))
    sds = jax.ShapeDtypeStruct((N, D), F32)
    sds_b = jax.ShapeDtypeStruct((NPAD, D), F32)
    return pl.pallas_call(
        _tables_body,
        grid=(NBLK,),
        in_specs=[blk, full((8, D)), blk, blk, full((D, D)), full((D, D)),
                  full((D, D)), full((1, D)), full((1, D))],
        out_specs=(blk, blk, blk),
        out_shape=(sds, sds, sds_b),
    )(h, st, p, bprev, w1, w2, w3, ci, cnext)


def _post_body(agg_ref, xs_ref, st_ref, w_ref, b_ref, h_ref, s1_ref, s2_ref,
               *, act):
    if act:
        st = st_ref[...]
        x = jnp.maximum(xs_ref[...] * st[0:1, :] + st[1:2, :], 0.0)
    else:
        x = xs_ref[...]
    m = agg_ref[...] + x
    h = jnp.dot(m, w_ref[...], preferred_element_type=F32) + b_ref[...]
    h_ref[...] = h

    @pl.when(pl.program_id(0) == 0)
    def _():
        s1_ref[...] = jnp.zeros_like(s1_ref)
        s2_ref[...] = jnp.zeros_like(s2_ref)

    s1b = jnp.sum(h, axis=0, keepdims=True)
    mb = s1b * F32(1.0 / RB)
    hc = h - mb
    s1_ref[0:1, :] += s1b
    s2_ref[0:1, :] += jnp.sum(hc * hc, axis=0, keepdims=True)
    s2_ref[1:2, :] += mb * mb


def _tc_post(aggp, xsrc, st, w, b, act):
    full = lambda s: pl.BlockSpec(s, lambda i: (0, 0))
    blk = pl.BlockSpec((RB, D), lambda i: (i, 0))
    return pl.pallas_call(
        functools.partial(_post_body, act=act),
        grid=(NBLK,),
        in_specs=[blk, blk, full((8, D)), full((D, D)), full((1, D))],
        out_specs=(blk, full((8, D)), full((8, D))),
        out_shape=(jax.ShapeDtypeStruct((N, D), F32),
                   jax.ShapeDtypeStruct((8, D), F32),
                   jax.ShapeDtypeStruct((8, D), F32)),
    )(aggp, xsrc, st, w, b)


def _stats_body(s1_ref, s2_ref, g_ref, bb_ref, st_ref):
    inv_n = F32(1.0 / N)
    mean = s1_ref[0:1, :] * inv_n
    var = (s2_ref[0:1, :] + F32(RB) * s2_ref[1:2, :]) * inv_n - mean * mean
    s = g_ref[...] * lax.rsqrt(var + 1e-5)
    t = bb_ref[...] - mean * s
    st_ref[...] = jnp.zeros_like(st_ref)
    st_ref[0:1, :] = s
    st_ref[1:2, :] = t


def _tc_stats(s1, s2, gamma, beta):
    full = lambda s: pl.BlockSpec(s, lambda i: (0, 0))
    return pl.pallas_call(
        _stats_body,
        grid=(1,),
        in_specs=[full((8, D)), full((8, D)), full((1, D)), full((1, D))],
        out_specs=full((8, D)),
        out_shape=jax.ShapeDtypeStruct((8, D), F32),
    )(s1, s2, gamma, beta)


def _pool_body(h_ref, st_ref, bid_ref, o_ref):
    st = st_ref[...]
    x = jnp.maximum(h_ref[...] * st[0:1, :] + st[1:2, :], 0.0)
    bid = bid_ref[0, 0, :]
    onehot = (lax.broadcasted_iota(I32, (RB, G), 1) == bid[:, None]).astype(F32)
    part = lax.dot_general(onehot, x, (((0,), (0,)), ((), ())),
                           preferred_element_type=F32)

    @pl.when(pl.program_id(0) == 0)
    def _():
        o_ref[...] = jnp.zeros_like(o_ref)

    o_ref[...] += part


def _tc_pool(h, st, bid3):
    full = lambda s: pl.BlockSpec(s, lambda i: (0, 0))
    return pl.pallas_call(
        _pool_body,
        grid=(NBLK,),
        in_specs=[pl.BlockSpec((RB, D), lambda i: (i, 0)), full((8, D)),
                  pl.BlockSpec((1, 1, RB), lambda i: (i, 0, 0))],
        out_specs=full((G, D)),
        out_shape=jax.ShapeDtypeStruct((G, D), F32),
    )(h, st, bid3)


def _out_body(p_ref, w_ref, b_ref, o_ref):
    o_ref[...] = (
        jnp.dot(p_ref[...], w_ref[...], preferred_element_type=F32) + b_ref[...]
    )


def _tc_out(pooled, w, b):
    full = lambda s: pl.BlockSpec(s, lambda i: (0, 0))
    return pl.pallas_call(
        _out_body,
        grid=(1,),
        in_specs=[full((G, D)), full((D, 128)), full((1, 128))],
        out_specs=full((G, 128)),
        out_shape=jax.ShapeDtypeStruct((G, 128), F32),
    )(pooled, w, b)


def _sc_mesh():
    return plsc.VectorSubcoreMesh(
        core_axis_name="c", subcore_axis_name="s",
        num_cores=NC, num_subcores=NS)


def _lane_i32(vec, lane, iota):
    return jnp.sum(jnp.where(iota == lane, vec, 0))


def _bin_body(row_h, col_h, w_h, brow_h, bcol_h, bw_h, offs_h,
              arow_h, acol_h, aw_h,
              brow_v, bcol_v, bw_v, inrow_v, incol_v, inw_v, ov_v):
    cid = lax.axis_index("c")
    sid = lax.axis_index("s")
    wid = sid * NC + cid
    base = wid * EPG
    iota = lax.iota(I32, L)

    def owner_body(c, carry):
        cursor, ov0, ov1 = carry
        ov0 = jnp.where(iota == c, cursor, ov0)
        ov1 = jnp.where(iota + L == c, cursor, ov1)

        def sb_body(sb, cur):
            off = pl.multiple_of(base + sb * SBB, 16)
            pltpu.sync_copy(row_h.at[pl.ds(off, SBB)], inrow_v)
            pltpu.sync_copy(col_h.at[pl.ds(off, SBB)], incol_v)
            pltpu.sync_copy(w_h.at[pl.ds(off, SBB)], inw_v)
            for j in range(SBB // L):
                cv = incol_v[pl.ds(j * L, L)]
                mask = ((cv >> 6) & 31) == c
                ones = jnp.where(mask, 1, 0)
                dest = cur + plsc.cumsum(ones) - 1
                plsc.store_scatter(brow_v, [dest],
                                   inrow_v[pl.ds(j * L, L)], mask=mask)
                plsc.store_scatter(bcol_v, [dest], cv, mask=mask)
                plsc.store_scatter(bw_v, [dest],
                                   inw_v[pl.ds(j * L, L)], mask=mask)
                cur = cur + jnp.sum(ones)
            return cur

        cursor = lax.fori_loop(0, NSB_BIN, sb_body, cursor)
        return cursor, ov0, ov1

    init = (I32(0), jnp.full((L,), EPG, I32), jnp.full((L,), EPG, I32))
    _, ov0, ov1 = lax.fori_loop(0, NW, owner_body, init)

    pltpu.sync_copy(brow_v.at[pl.ds(0, EPG)], arow_h.at[pl.ds(base, EPG)])
    pltpu.sync_copy(bcol_v.at[pl.ds(0, EPG)], acol_h.at[pl.ds(base, EPG)])
    pltpu.sync_copy(bw_v.at[pl.ds(0, EPG)], aw_h.at[pl.ds(base, EPG)])

    def seg_body(o, cursor):
        lane_a = o & 15
        vsel_a = o >> 4
        s_a = _lane_i32(ov0, lane_a, iota)
        s_b = _lane_i32(ov1, lane_a, iota)
        seg_lo = jnp.where(vsel_a == 0, s_a, s_b)
        o2 = o + 1
        lane_b = o2 & 15
        vsel_b = o2 >> 4
        e_a = _lane_i32(ov0, lane_b, iota)
        e_b = _lane_i32(ov1, lane_b, iota)
        seg_hi = jnp.where(vsel_b == 0, e_a,
                           jnp.where(vsel_b == 1, e_b, EPG))
        astart = seg_lo & ~15
        span = seg_hi - astart
        nsb = (span + SBB - 1) // SBB

        def z_body(z, carry):
            cur, ovz = carry
            ovz = jnp.where(iota == z, cur, ovz)

            def sb_body(sb, cur2):
                abase = pl.multiple_of(astart + sb * SBB, 16)
                off = pl.multiple_of(base + abase, 16)
                pltpu.sync_copy(arow_h.at[pl.ds(off, SBB)], inrow_v)
                pltpu.sync_copy(acol_h.at[pl.ds(off, SBB)], incol_v)
                pltpu.sync_copy(aw_h.at[pl.ds(off, SBB)], inw_v)
                for j in range(SBB // L):
                    pos = abase + j * L + iota
                    valid = (pos >= seg_lo) & (pos < seg_hi)
                    cv = incol_v[pl.ds(j * L, L)]
                    mask = valid & ((cv >> 12) == z)
                    ones = jnp.where(mask, 1, 0)
                    dest = cur2 + plsc.cumsum(ones) - 1
                    plsc.store_scatter(brow_v, [dest],
                                       inrow_v[pl.ds(j * L, L)], mask=mask)
                    plsc.store_scatter(bcol_v, [dest], cv, mask=mask)
                    plsc.store_scatter(bw_v, [dest],
                                       inw_v[pl.ds(j * L, L)], mask=mask)
                    cur2 = cur2 + jnp.sum(ones)
                return cur2

            cur = lax.fori_loop(0, nsb, sb_body, cur)
            return cur, ovz

        cursor, ovz = lax.fori_loop(
            0, NZ, z_body, (cursor, jnp.full((L,), EPG, I32)))
        ovz = jnp.where(iota == NZ, cursor, ovz)
        ov_v[pl.ds(0, L)] = ovz
        dst = pl.multiple_of((wid * NW + o) * L, 16)
        pltpu.sync_copy(ov_v.at[pl.ds(0, L)], offs_h.at[pl.ds(dst, L)])
        return cursor

    lax.fori_loop(0, NW, seg_body, I32(0))

    pltpu.sync_copy(brow_v.at[pl.ds(0, EPG)], brow_h.at[pl.ds(base, EPG)])
    pltpu.sync_copy(bcol_v.at[pl.ds(0, EPG)], bcol_h.at[pl.ds(base, EPG)])
    pltpu.sync_copy(bw_v.at[pl.ds(0, EPG)], bw_h.at[pl.ds(base, EPG)])


def _sc_bin(row_p, col_p, w_p):
    f = pl.kernel(
        _bin_body,
        out_type=(
            jax.ShapeDtypeStruct((ETOT + SB,), I32),
            jax.ShapeDtypeStruct((ETOT + SB,), I32),
            jax.ShapeDtypeStruct((ETOT + SB,), F32),
            jax.ShapeDtypeStruct((NW * NW * L,), I32),
            jax.ShapeDtypeStruct((ETOT + SBB,), I32),
            jax.ShapeDtypeStruct((ETOT + SBB,), I32),
            jax.ShapeDtypeStruct((ETOT + SBB,), F32),
        ),
        mesh=_sc_mesh(),
        compiler_params=pltpu.CompilerParams(needs_layout_passes=False),
        scratch_types=[
            pltpu.VMEM((EPG + L,), I32),
            pltpu.VMEM((EPG + L,), I32),
            pltpu.VMEM((EPG + L,), F32),
            pltpu.VMEM((SBB,), I32),
            pltpu.VMEM((SBB,), I32),
            pltpu.VMEM((SBB,), F32),
            pltpu.VMEM((L,), I32),
        ],
    )
    return f(row_p, col_p, w_p)


def _scatter_body(a_h, b_h, u_h, brow_h, bcol_h, bw_h, offs_h, agg_h,
                  acc, mbuf, bwin, vrow, vcol, vw, rowidx, relbuf,
                  u_v, offs_v, sem_a, sem_b):
    cid = lax.axis_index("c")
    sid = lax.axis_index("s")
    wid = sid * NC + cid
    iota = lax.iota(I32, L)

    pltpu.sync_copy(u_h, u_v)
    u_regs = [u_v[pl.ds(j * L, L)] for j in range(D // L)]
    lane_j = [j * L + iota for j in range(D // L)]

    def z_body(z, _):
        def zr(r, _):
            for j in range(D // L):
                acc[r, pl.ds(j * L, L)] = jnp.zeros((L,), F32)
            return 0

        lax.fori_loop(0, ACC_ROWS, zr, 0)

        for s in range(2):
            srcb = pl.multiple_of(z * CHUNK + s * 2048 + wid * 64, 64)
            pltpu.sync_copy(b_h.at[pl.ds(srcb, 64)],
                            bwin.at[pl.ds(s * 64, 64)])

        def tile_body(t, _):
            src = pl.multiple_of((t * NW + wid) * L, 16)
            pltpu.sync_copy(offs_h.at[pl.ds(src, L)], offs_v)
            ov = offs_v[...]
            start = _lane_i32(ov, z, iota)
            end = _lane_i32(ov, z + 1, iota)
            count = end - start
            astart = start & ~15
            span = start + count - astart
            nsb = (span + SB - 1) // SB
            base_e = t * EPG

            def sb_body(sb, _):
                abase = pl.multiple_of(astart + sb * SB, 16)
                pltpu.sync_copy(brow_h.at[pl.ds(base_e + abase, SB)], vrow)
                pltpu.sync_copy(bcol_h.at[pl.ds(base_e + abase, SB)], vcol)
                pltpu.sync_copy(bw_h.at[pl.ds(base_e + abase, SB)], vw)
                for j in range(SB // L):
                    pos = abase + j * L + iota
                    valid = (pos >= start) & (pos < start + count)
                    cv = vcol[pl.ds(j * L, L)]
                    rowidx[pl.ds(j * L, L)] = jnp.where(
                        valid, vrow[pl.ds(j * L, L)], 0)
                    relbuf[pl.ds(j * L, L)] = ((cv >> 11) & 1) * 64 + (cv & 63)
                pltpu.async_copy(a_h.at[rowidx], mbuf, sem_a).wait()

                e_lo = jnp.maximum(start - abase, 0)
                e_hi = jnp.minimum(start + count - abase, SB)

                def e_body(e, _):
                    e16 = jnp.full((L,), e, I32)
                    we = plsc.load_gather(vw, [e16])
                    rel16 = plsc.load_gather(relbuf, [e16])
                    for j in range(D // L):
                        a = mbuf[e, pl.ds(j * L, L)]
                        b = plsc.load_gather(bwin, [rel16, lane_j[j]])
                        msg = jnp.maximum(a + b + we * u_regs[j], 0.0)
                        plsc.addupdate_scatter(acc, [rel16, lane_j[j]], msg)
                    return 0

                lax.fori_loop(e_lo, e_hi, e_body, 0)
                return 0

            lax.fori_loop(0, nsb, sb_body, 0)
            return 0

        lax.fori_loop(0, NW, tile_body, 0)

        for s in range(2):
            dsts = z * CHUNK + s * 2048 + wid * 64
            dst = pl.multiple_of(dsts, 64)
            pltpu.sync_copy(acc.at[pl.ds(s * 64, 64)],
                            agg_h.at[pl.ds(dst, 64)])
        return 0

    lax.fori_loop(0, NZ, z_body, 0)


def _sc_scatter(a, b, u, brow, bcol, bw, offs):
    f = pl.kernel(
        _scatter_body,
        out_type=jax.ShapeDtypeStruct((NPAD, D), F32),
        mesh=_sc_mesh(),
        compiler_params=pltpu.CompilerParams(needs_layout_passes=False),
        scratch_types=[
            pltpu.VMEM((ACC_ROWS, D), F32),
            pltpu.VMEM((SB, D), F32),
            pltpu.VMEM((128, D), F32),
            pltpu.VMEM((SB,), I32),
            pltpu.VMEM((SB,), I32),
            pltpu.VMEM((SB,), F32),
            pltpu.VMEM((SB,), I32),
            pltpu.VMEM((SB,), I32),
            pltpu.VMEM((D,), F32),
            pltpu.VMEM((L,), I32),
            pltpu.SemaphoreType.DMA,
            pltpu.SemaphoreType.DMA,
        ],
    )
    return f(a, b, u, brow, bcol, bw, offs)


def kernel(node_features, edge_index, atom_weights, batch, atom_W, atom_b,
           edge_W, edge_b, mlp_W, mlp_b, nn_W, nn_b, bn_gamma, bn_beta,
           out_W, out_b):
    nf_p = jnp.pad(node_features, ((0, 0), (0, 128 - node_features.shape[1])))
    atom_W_p = jnp.pad(atom_W, ((0, 128 - atom_W.shape[0]), (0, 0)))
    row = edge_index[0].astype(I32)
    col = edge_index[1].astype(I32)
    w_e = atom_weights[:, 0]
    row_p = jnp.pad(row, (0, ETOT - E))
    col_p = jnp.pad(col, (0, ETOT - E), constant_values=PAD_COL)
    w_p = jnp.pad(w_e, (0, ETOT - E))
    w1 = mlp_W[0:D]
    w2 = mlp_W[D:2 * D]
    w3 = mlp_W[2 * D:3 * D]
    m0 = jnp.concatenate([edge_W, edge_b[None, :], jnp.zeros((6, D), F32)], 0)
    badd = jnp.concatenate(
        [jnp.zeros((1, D), F32), mlp_b[None, :], jnp.zeros((6, D), F32)], 0)
    bid3 = batch.astype(I32).reshape(NBLK, 1, RB)
    out_W_p = jnp.pad(out_W, ((0, 0), (0, 128 - out_W.shape[1])))
    out_b_p = jnp.pad(out_b, (0, 128 - out_b.shape[0]))[None, :]

    x0 = _tc_embed(nf_p, atom_W_p, atom_b[None, :])
    uc = _tc_uc(m0, badd, w3)
    brow, bcol, bw, offs, _, _, _ = _sc_bin(row_p, col_p, w_p)

    p, a, b = _tc_tables_first(x0, w1, w2, uc[1:2])
    agg = _sc_scatter(a, b, uc[0], brow, bcol, bw, offs)
    h, s1, s2 = _tc_post(agg, x0, uc, nn_W, nn_b[None, :], act=False)
    st = _tc_stats(s1, s2, bn_gamma[None, :], bn_beta[None, :])

    for i in (1, 2, 3):
        ci = uc[2 * i - 1:2 * i]
        cn = uc[2 * i + 1:2 * i + 2]
        p, a, b = _tc_tables(h, st, p, b, w1, w2, w3, ci, cn)
        agg = _sc_scatter(a, b, uc[2 * i], brow, bcol, bw, offs)
        h, s1, s2 = _tc_post(agg, h, st, nn_W, nn_b[None, :], act=True)
        st = _tc_stats(s1, s2, bn_gamma[None, :], bn_beta[None, :])

    pooled = _tc_pool(h, st, bid3)
    out = _tc_out(pooled, out_W_p, out_b_p)
    return out[:, :6]


# speedup vs baseline: 3.4679x; 1.4318x over previous
import functools

import jax
import jax.numpy as jnp
from jax import lax
from jax.experimental import pallas as pl
from jax.experimental.pallas import tpu as pltpu
from jax.experimental.pallas import tpu_sc as plsc

N = 50000
E = 800000
G = 256
D = 256
F32 = jnp.float32
I32 = jnp.int32

RB = 2000
NBLK = N // RB

NC = 2
NS = 16
L = 16
NW = NC * NS

EPG = 25088
ETOT = NW * EPG
SBB = 512
NSB_BIN = EPG // SBB
SB = 16
NPAD = 53248
NZ = 13
CHUNK = 4096
ACC_ROWS = 136
PAD_COL = NPAD - 1


def _embed_body(x_ref, w_ref, b_ref, o_ref):
    o_ref[...] = (
        jnp.dot(x_ref[...], w_ref[...], preferred_element_type=F32) + b_ref[...]
    )


def _tc_embed(x, w, b):
    return pl.pallas_call(
        _embed_body,
        grid=(NBLK,),
        in_specs=[
            pl.BlockSpec((RB, 128), lambda i: (i, 0)),
            pl.BlockSpec((128, D), lambda i: (0, 0)),
            pl.BlockSpec((1, D), lambda i: (0, 0)),
        ],
        out_specs=pl.BlockSpec((RB, D), lambda i: (i, 0)),
        out_shape=jax.ShapeDtypeStruct((N, D), F32),
    )(x, w, b)


def _uc_body(m0_ref, badd_ref, w3_ref, uc_ref):
    m = m0_ref[...]
    badd = badd_ref[...]
    w3 = w3_ref[...]
    for k in range(4):
        m = jnp.dot(m, w3, preferred_element_type=F32) + badd
        uc_ref[pl.ds(2 * k, 2), :] = m[0:2, :]


def _tc_uc(m0, badd, w3):
    return pl.pallas_call(
        _uc_body,
        grid=(1,),
        in_specs=[
            pl.BlockSpec((8, D), lambda i: (0, 0)),
            pl.BlockSpec((8, D), lambda i: (0, 0)),
            pl.BlockSpec((D, D), lambda i: (0, 0)),
        ],
        out_specs=pl.BlockSpec((8, D), lambda i: (0, 0)),
        out_shape=jax.ShapeDtypeStruct((8, D), F32),
    )(m0, badd, w3)


def _tables_first_body(x_ref, w1_ref, w2_ref, c1_ref, p_ref, a_ref, b_ref):
    x = x_ref[...]
    p1 = jnp.dot(x, w1_ref[...], preferred_element_type=F32)
    q1 = jnp.dot(x, w2_ref[...], preferred_element_type=F32)
    p_ref[...] = p1
    a_ref[...] = x + p1
    b_ref[...] = q1 + c1_ref[...]


def _tc_tables_first(x, w1, w2, c1):
    full = lambda s: pl.BlockSpec(s, lambda i: (0, 0))
    blk = pl.BlockSpec((RB, D), lambda i: (i, 0))
    sds = jax.ShapeDtypeStruct((N, D), F32)
    sds_b = jax.ShapeDtypeStruct((NPAD, D), F32)
    return pl.pallas_call(
        _tables_first_body,
        grid=(NBLK,),
        in_specs=[blk, full((D, D)), full((D, D)), full((1, D))],
        out_specs=(blk, blk, blk),
        out_shape=(sds, sds, sds_b),
    )(x, w1, w2, c1)


def _tables_body(h_ref, st_ref, p_ref, bp_ref, w1_ref, w2_ref, w3_ref,
                 ci_ref, cn_ref, po_ref, a_ref, b_ref):
    st = st_ref[...]
    x = jnp.maximum(h_ref[...] * st[0:1, :] + st[1:2, :], 0.0)
    q = bp_ref[...] - ci_ref[...]
    pn = (jnp.dot(x, w1_ref[...], preferred_element_type=F32)
          + jnp.dot(p_ref[...], w3_ref[...], preferred_element_type=F32))
    qn = (jnp.dot(x, w2_ref[...], preferred_element_type=F32)
          + jnp.dot(q, w3_ref[...], preferred_element_type=F32))
    po_ref[...] = pn
    a_ref[...] = x + pn
    b_ref[...] = qn + cn_ref[...]


def _tc_tables(h, st, p, bprev, w1, w2, w3, ci, cnext):
    full = lambda s: pl.BlockSpec(s, lambda i: (0, 0))
    blk = pl.BlockSpec((RB, D), lambda i: (i, 0))
    sds = jax.ShapeDtypeStruct((N, D), F32)
    sds_b = jax.ShapeDtypeStruct((NPAD, D), F32)
    return pl.pallas_call(
        _tables_body,
        grid=(NBLK,),
        in_specs=[blk, full((8, D)), blk, blk, full((D, D)), full((D, D)),
                  full((D, D)), full((1, D)), full((1, D))],
        out_specs=(blk, blk, blk),
        out_shape=(sds, sds, sds_b),
    )(h, st, p, bprev, w1, w2, w3, ci, cnext)


def _post_body(agg_ref, xs_ref, st_ref, w_ref, b_ref, h_ref, s1_ref, s2_ref,
               *, act):
    if act:
        st = st_ref[...]
        x = jnp.maximum(xs_ref[...] * st[0:1, :] + st[1:2, :], 0.0)
    else:
        x = xs_ref[...]
    m = agg_ref[...] + x
    h = jnp.dot(m, w_ref[...], preferred_element_type=F32) + b_ref[...]
    h_ref[...] = h

    @pl.when(pl.program_id(0) == 0)
    def _():
        s1_ref[...] = jnp.zeros_like(s1_ref)
        s2_ref[...] = jnp.zeros_like(s2_ref)

    s1b = jnp.sum(h, axis=0, keepdims=True)
    mb = s1b * F32(1.0 / RB)
    hc = h - mb
    s1_ref[0:1, :] += s1b
    s2_ref[0:1, :] += jnp.sum(hc * hc, axis=0, keepdims=True)
    s2_ref[1:2, :] += mb * mb


def _tc_post(aggp, xsrc, st, w, b, act):
    full = lambda s: pl.BlockSpec(s, lambda i: (0, 0))
    blk = pl.BlockSpec((RB, D), lambda i: (i, 0))
    return pl.pallas_call(
        functools.partial(_post_body, act=act),
        grid=(NBLK,),
        in_specs=[blk, blk, full((8, D)), full((D, D)), full((1, D))],
        out_specs=(blk, full((8, D)), full((8, D))),
        out_shape=(jax.ShapeDtypeStruct((N, D), F32),
                   jax.ShapeDtypeStruct((8, D), F32),
                   jax.ShapeDtypeStruct((8, D), F32)),
    )(aggp, xsrc, st, w, b)


def _stats_body(s1_ref, s2_ref, g_ref, bb_ref, st_ref):
    inv_n = F32(1.0 / N)
    mean = s1_ref[0:1, :] * inv_n
    var = (s2_ref[0:1, :] + F32(RB) * s2_ref[1:2, :]) * inv_n - mean * mean
    s = g_ref[...] * lax.rsqrt(var + 1e-5)
    t = bb_ref[...] - mean * s
    st_ref[...] = jnp.zeros_like(st_ref)
    st_ref[0:1, :] = s
    st_ref[1:2, :] = t


def _tc_stats(s1, s2, gamma, beta):
    full = lambda s: pl.BlockSpec(s, lambda i: (0, 0))
    return pl.pallas_call(
        _stats_body,
        grid=(1,),
        in_specs=[full((8, D)), full((8, D)), full((1, D)), full((1, D))],
        out_specs=full((8, D)),
        out_shape=jax.ShapeDtypeStruct((8, D), F32),
    )(s1, s2, gamma, beta)


def _pool_body(h_ref, st_ref, bid_ref, o_ref):
    st = st_ref[...]
    x = jnp.maximum(h_ref[...] * st[0:1, :] + st[1:2, :], 0.0)
    bid = bid_ref[0, 0, :]
    onehot = (lax.broadcasted_iota(I32, (RB, G), 1) == bid[:, None]).astype(F32)
    part = lax.dot_general(onehot, x, (((0,), (0,)), ((), ())),
                           preferred_element_type=F32)

    @pl.when(pl.program_id(0) == 0)
    def _():
        o_ref[...] = jnp.zeros_like(o_ref)

    o_ref[...] += part


def _tc_pool(h, st, bid3):
    full = lambda s: pl.BlockSpec(s, lambda i: (0, 0))
    return pl.pallas_call(
        _pool_body,
        grid=(NBLK,),
        in_specs=[pl.BlockSpec((RB, D), lambda i: (i, 0)), full((8, D)),
                  pl.BlockSpec((1, 1, RB), lambda i: (i, 0, 0))],
        out_specs=full((G, D)),
        out_shape=jax.ShapeDtypeStruct((G, D), F32),
    )(h, st, bid3)


def _out_body(p_ref, w_ref, b_ref, o_ref):
    o_ref[...] = (
        jnp.dot(p_ref[...], w_ref[...], preferred_element_type=F32) + b_ref[...]
    )


def _tc_out(pooled, w, b):
    full = lambda s: pl.BlockSpec(s, lambda i: (0, 0))
    return pl.pallas_call(
        _out_body,
        grid=(1,),
        in_specs=[full((G, D)), full((D, 128)), full((1, 128))],
        out_specs=full((G, 128)),
        out_shape=jax.ShapeDtypeStruct((G, 128), F32),
    )(pooled, w, b)


def _sc_mesh():
    return plsc.VectorSubcoreMesh(
        core_axis_name="c", subcore_axis_name="s",
        num_cores=NC, num_subcores=NS)


def _lane_i32(vec, lane, iota):
    return jnp.sum(jnp.where(iota == lane, vec, 0))


def _bin_body(row_h, col_h, w_h, brow_h, bcol_h, bw_h, offs_h,
              arow_h, acol_h, aw_h,
              brow_v, bcol_v, bw_v, inrow_v, incol_v, inw_v, ov_v):
    cid = lax.axis_index("c")
    sid = lax.axis_index("s")
    wid = sid * NC + cid
    base = wid * EPG
    iota = lax.iota(I32, L)

    def owner_body(c, carry):
        cursor, ov0, ov1 = carry
        ov0 = jnp.where(iota == c, cursor, ov0)
        ov1 = jnp.where(iota + L == c, cursor, ov1)

        def sb_body(sb, cur):
            off = pl.multiple_of(base + sb * SBB, 16)
            pltpu.sync_copy(row_h.at[pl.ds(off, SBB)], inrow_v)
            pltpu.sync_copy(col_h.at[pl.ds(off, SBB)], incol_v)
            pltpu.sync_copy(w_h.at[pl.ds(off, SBB)], inw_v)
            for j in range(SBB // L):
                cv = incol_v[pl.ds(j * L, L)]
                mask = ((cv >> 6) & 31) == c
                ones = jnp.where(mask, 1, 0)
                dest = cur + plsc.cumsum(ones) - 1
                plsc.store_scatter(brow_v, [dest],
                                   inrow_v[pl.ds(j * L, L)], mask=mask)
                plsc.store_scatter(bcol_v, [dest], cv, mask=mask)
                plsc.store_scatter(bw_v, [dest],
                                   inw_v[pl.ds(j * L, L)], mask=mask)
                cur = cur + jnp.sum(ones)
            return cur

        cursor = lax.fori_loop(0, NSB_BIN, sb_body, cursor)
        return cursor, ov0, ov1

    init = (I32(0), jnp.full((L,), EPG, I32), jnp.full((L,), EPG, I32))
    _, ov0, ov1 = lax.fori_loop(0, NW, owner_body, init)

    pltpu.sync_copy(brow_v.at[pl.ds(0, EPG)], arow_h.at[pl.ds(base, EPG)])
    pltpu.sync_copy(bcol_v.at[pl.ds(0, EPG)], acol_h.at[pl.ds(base, EPG)])
    pltpu.sync_copy(bw_v.at[pl.ds(0, EPG)], aw_h.at[pl.ds(base, EPG)])

    def seg_body(o, cursor):
        lane_a = o & 15
        vsel_a = o >> 4
        s_a = _lane_i32(ov0, lane_a, iota)
        s_b = _lane_i32(ov1, lane_a, iota)
        seg_lo = jnp.where(vsel_a == 0, s_a, s_b)
        o2 = o + 1
        lane_b = o2 & 15
        vsel_b = o2 >> 4
        e_a = _lane_i32(ov0, lane_b, iota)
        e_b = _lane_i32(ov1, lane_b, iota)
        seg_hi = jnp.where(vsel_b == 0, e_a,
                           jnp.where(vsel_b == 1, e_b, EPG))
        astart = seg_lo & ~15
        span = seg_hi - astart
        nsb = (span + SBB - 1) // SBB

        def z_body(z, carry):
            cur, ovz = carry
            ovz = jnp.where(iota == z, cur, ovz)

            def sb_body(sb, cur2):
                abase = pl.multiple_of(astart + sb * SBB, 16)
                off = pl.multiple_of(base + abase, 16)
                pltpu.sync_copy(arow_h.at[pl.ds(off, SBB)], inrow_v)
                pltpu.sync_copy(acol_h.at[pl.ds(off, SBB)], incol_v)
                pltpu.sync_copy(aw_h.at[pl.ds(off, SBB)], inw_v)
                for j in range(SBB // L):
                    pos = abase + j * L + iota
                    valid = (pos >= seg_lo) & (pos < seg_hi)
                    cv = incol_v[pl.ds(j * L, L)]
                    mask = valid & ((cv >> 12) == z)
                    ones = jnp.where(mask, 1, 0)
                    dest = cur2 + plsc.cumsum(ones) - 1
                    plsc.store_scatter(brow_v, [dest],
                                       inrow_v[pl.ds(j * L, L)], mask=mask)
                    plsc.store_scatter(bcol_v, [dest], cv, mask=mask)
                    plsc.store_scatter(bw_v, [dest],
                                       inw_v[pl.ds(j * L, L)], mask=mask)
                    cur2 = cur2 + jnp.sum(ones)
                return cur2

            cur = lax.fori_loop(0, nsb, sb_body, cur)
            return cur, ovz

        cursor, ovz = lax.fori_loop(
            0, NZ, z_body, (cursor, jnp.full((L,), EPG, I32)))
        ovz = jnp.where(iota == NZ, cursor, ovz)
        ov_v[pl.ds(0, L)] = ovz
        dst = pl.multiple_of((wid * NW + o) * L, 16)
        pltpu.sync_copy(ov_v.at[pl.ds(0, L)], offs_h.at[pl.ds(dst, L)])
        return cursor

    lax.fori_loop(0, NW, seg_body, I32(0))

    pltpu.sync_copy(brow_v.at[pl.ds(0, EPG)], brow_h.at[pl.ds(base, EPG)])
    pltpu.sync_copy(bcol_v.at[pl.ds(0, EPG)], bcol_h.at[pl.ds(base, EPG)])
    pltpu.sync_copy(bw_v.at[pl.ds(0, EPG)], bw_h.at[pl.ds(base, EPG)])


def _sc_bin(row_p, col_p, w_p):
    f = pl.kernel(
        _bin_body,
        out_type=(
            jax.ShapeDtypeStruct((ETOT + SB,), I32),
            jax.ShapeDtypeStruct((ETOT + SB,), I32),
            jax.ShapeDtypeStruct((ETOT + SB,), F32),
            jax.ShapeDtypeStruct((NW * NW * L,), I32),
            jax.ShapeDtypeStruct((ETOT + SBB,), I32),
            jax.ShapeDtypeStruct((ETOT + SBB,), I32),
            jax.ShapeDtypeStruct((ETOT + SBB,), F32),
        ),
        mesh=_sc_mesh(),
        compiler_params=pltpu.CompilerParams(needs_layout_passes=False),
        scratch_types=[
            pltpu.VMEM((EPG + L,), I32),
            pltpu.VMEM((EPG + L,), I32),
            pltpu.VMEM((EPG + L,), F32),
            pltpu.VMEM((SBB,), I32),
            pltpu.VMEM((SBB,), I32),
            pltpu.VMEM((SBB,), F32),
            pltpu.VMEM((L,), I32),
        ],
    )
    return f(row_p, col_p, w_p)


def _scatter_body(a_h, b_h, u_h, brow_h, bcol_h, bw_h, offs_h, agg_h,
                  acc, mbuf, bwin, vrow, vcol, vw, rowidx, relbuf,
                  u_v, offs_v, sem_a, sem_b):
    cid = lax.axis_index("c")
    sid = lax.axis_index("s")
    wid = sid * NC + cid
    iota = lax.iota(I32, L)

    pltpu.sync_copy(u_h, u_v)
    u_regs = [u_v[pl.ds(j * L, L)] for j in range(D // L)]
    lane_j = [j * L + iota for j in range(D // L)]

    def z_body(z, _):
        def zr(r, _):
            for j in range(D // L):
                acc[r, pl.ds(j * L, L)] = jnp.zeros((L,), F32)
            return 0

        lax.fori_loop(0, ACC_ROWS, zr, 0)

        for s in range(2):
            srcb = pl.multiple_of(z * CHUNK + s * 2048 + wid * 64, 64)
            pltpu.sync_copy(b_h.at[pl.ds(srcb, 64)],
                            bwin.at[pl.ds(s * 64, 64)])

        def tile_body(t, _):
            src = pl.multiple_of((t * NW + wid) * L, 16)
            pltpu.sync_copy(offs_h.at[pl.ds(src, L)], offs_v)
            ov = offs_v[...]
            start = _lane_i32(ov, z, iota)
            end = _lane_i32(ov, z + 1, iota)
            count = end - start
            astart = start & ~15
            span = start + count - astart
            nsb = (span + SB - 1) // SB
            base_e = t * EPG

            def sb_body(sb, _):
                abase = pl.multiple_of(astart + sb * SB, 16)
                pltpu.sync_copy(brow_h.at[pl.ds(base_e + abase, SB)], vrow)
                pltpu.sync_copy(bcol_h.at[pl.ds(base_e + abase, SB)], vcol)
                pltpu.sync_copy(bw_h.at[pl.ds(base_e + abase, SB)], vw)
                for j in range(SB // L):
                    pos = abase + j * L + iota
                    valid = (pos >= start) & (pos < start + count)
                    cv = vcol[pl.ds(j * L, L)]
                    rowidx[pl.ds(j * L, L)] = jnp.where(
                        valid, vrow[pl.ds(j * L, L)], 0)
                    relbuf[pl.ds(j * L, L)] = ((cv >> 11) & 1) * 64 + (cv & 63)
                pltpu.async_copy(a_h.at[rowidx], mbuf, sem_a).wait()

                e_lo = jnp.maximum(start - abase, 0)
                e_hi = jnp.minimum(start + count - abase, SB)

                def e_body(e, _):
                    e16 = jnp.full((L,), e, I32)
                    we = plsc.load_gather(vw, [e16])
                    rel16 = plsc.load_gather(relbuf, [e16])
                    for j in range(D // L):
                        a = mbuf[e, pl.ds(j * L, L)]
                        b = plsc.load_gather(bwin, [rel16, lane_j[j]])
                        msg = jnp.maximum(a + b + we * u_regs[j], 0.0)
                        plsc.addupdate_scatter(acc, [rel16, lane_j[j]], msg)
                    return 0

                lax.fori_loop(e_lo, e_hi, e_body, 0)
                return 0

            lax.fori_loop(0, nsb, sb_body, 0)
            return 0

        lax.fori_loop(0, NW, tile_body, 0)

        for s in range(2):
            dsts = z * CHUNK + s * 2048 + wid * 64
            dst = pl.multiple_of(dsts, 64)
            pltpu.sync_copy(acc.at[pl.ds(s * 64, 64)],
                            agg_h.at[pl.ds(dst, 64)])
        return 0

    lax.fori_loop(0, NZ, z_body, 0)


def _sc_scatter(a, b, u, brow, bcol, bw, offs):
    f = pl.kernel(
        _scatter_body,
        out_type=jax.ShapeDtypeStruct((NPAD, D), F32),
        mesh=_sc_mesh(),
        compiler_params=pltpu.CompilerParams(needs_layout_passes=False),
        scratch_types=[
            pltpu.VMEM((ACC_ROWS, D), F32),
            pltpu.VMEM((SB, D), F32),
            pltpu.VMEM((128, D), F32),
            pltpu.VMEM((SB,), I32),
            pltpu.VMEM((SB,), I32),
            pltpu.VMEM((SB,), F32),
            pltpu.VMEM((SB,), I32),
            pltpu.VMEM((SB,), I32),
            pltpu.VMEM((D,), F32),
            pltpu.VMEM((L,), I32),
            pltpu.SemaphoreType.DMA,
            pltpu.SemaphoreType.DMA,
        ],
    )
    return f(a, b, u, brow, bcol, bw, offs)


def kernel(node_features, edge_index, atom_weights, batch, atom_W, atom_b,
           edge_W, edge_b, mlp_W, mlp_b, nn_W, nn_b, bn_gamma, bn_beta,
           out_W, out_b):
    nf_p = jnp.pad(node_features, ((0, 0), (0, 128 - node_features.shape[1])))
    atom_W_p = jnp.pad(atom_W, ((0, 128 - atom_W.shape[0]), (0, 0)))
    row = edge_index[0].astype(I32)
    col = edge_index[1].astype(I32)
    w_e = atom_weights[:, 0]
    row_p = jnp.pad(row, (0, ETOT - E))
    col_p = jnp.pad(col, (0, ETOT - E), constant_values=PAD_COL)
    w_p = jnp.pad(w_e, (0, ETOT - E))
    w1 = mlp_W[0:D]
    w2 = mlp_W[D:2 * D]
    w3 = mlp_W[2 * D:3 * D]
    m0 = jnp.concatenate([edge_W, edge_b[None, :], jnp.zeros((6, D), F32)], 0)
    badd = jnp.concatenate(
        [jnp.zeros((1, D), F32), mlp_b[None, :], jnp.zeros((6, D), F32)], 0)
    bid3 = batch.astype(I32).reshape(NBLK, 1, RB)
    out_W_p = jnp.pad(out_W, ((0, 0), (0, 128 - out_W.shape[1])))
    out_b_p = jnp.pad(out_b, (0, 128 - out_b.shape[0]))[None, :]

    x0 = _tc_embed(nf_p, atom_W_p, atom_b[None, :])
    uc = _tc_uc(m0, badd, w3)
    brow, bcol, bw, offs, _, _, _ = _sc_bin(row_p, col_p, w_p)

    p, a, b = _tc_tables_first(x0, w1, w2, uc[1:2])
    agg = _sc_scatter(a, b, uc[0], brow, bcol, bw, offs)
    h, s1, s2 = _tc_post(agg, x0, uc, nn_W, nn_b[None, :], act=False)
    st = _tc_stats(s1, s2, bn_gamma[None, :], bn_beta[None, :])

    for i in (1, 2, 3):
        ci = uc[2 * i - 1:2 * i]
        cn = uc[2 * i + 1:2 * i + 2]
        p, a, b = _tc_tables(h, st, p, b, w1, w2, w3, ci, cn)
        agg = _sc_scatter(a, b, uc[2 * i], brow, bcol, bw, offs)
        h, s1, s2 = _tc_post(agg, h, st, nn_W, nn_b[None, :], act=True)
        st = _tc_stats(s1, s2, bn_gamma[None, :], bn_beta[None, :])

    pooled = _tc_pool(h, st, bid3)
    out = _tc_out(pooled, out_W_p, out_b_p)
    return out[:, :6]
